# Initial kernel scaffold; baseline (speedup 1.0000x reference)
import jax
import jax.numpy as jnp
from jax.experimental import pallas as pl


def kernel(x, adj):
    raise NotImplementedError("write your pallas kernel here")


# speedup vs baseline: 6.0817x; 6.0817x over previous
import jax
import jax.numpy as jnp
from jax import lax
from jax.experimental import pallas as pl
from jax.experimental.pallas import tpu as pltpu
from jax.experimental.pallas import tpu_sc as plsc

_MIN_NORM = 1e-15
_MAXNORM = 1.0 - 4e-3

_N, _E, _D = 10000, 160000, 256
_HALF = _D // 2
_LANES = 128
_CHUNK = 2 * _LANES
_NCHUNKS = _E // _CHUNK
_NSUB = 16
_CPR = (_N // _NSUB) // 8 * 8
_TAIL = _N - _CPR * _NSUB
_BN = 1000


def _logmap_body(x_ref, o_ref):
    x = x_ref[...]
    nrm = jnp.sqrt(jnp.sum(x * x, axis=1, keepdims=True))
    nrm = jnp.maximum(nrm, _MIN_NORM)
    t = jnp.clip(nrm, -1.0 + 1e-7, 1.0 - 1e-7)
    art = 0.5 * (jnp.log1p(t) - jnp.log1p(-t))
    xt = x * (art / nrm)
    o_ref[0] = xt[:, :_HALF]
    o_ref[1] = xt[:, _HALF:]


def _expmap_body(s_ref, o_ref):
    lo = s_ref[0]
    hi = s_ref[1]
    nrm = jnp.sqrt(jnp.sum(lo * lo, axis=1, keepdims=True)
                   + jnp.sum(hi * hi, axis=1, keepdims=True))
    nrm = jnp.maximum(nrm, _MIN_NORM)
    g = jnp.tanh(nrm) / nrm
    ylo = lo * g
    yhi = hi * g
    ynrm = jnp.sqrt(jnp.sum(ylo * ylo, axis=1, keepdims=True)
                    + jnp.sum(yhi * yhi, axis=1, keepdims=True))
    ynrm = jnp.maximum(ynrm, _MIN_NORM)
    scale = jnp.where(ynrm > _MAXNORM, _MAXNORM / ynrm, 1.0)
    o_ref[:, :_HALF] = ylo * scale
    o_ref[:, _HALF:] = yhi * scale


def _sc_body(table, idx_hbm, zeros_hbm, out_hbm, idx_v, rows0, rows1, acc, sem):
    c = lax.axis_index("c")
    s = lax.axis_index("s")

    base = s * _CPR
    pltpu.sync_copy(zeros_hbm, acc.at[pl.ds(base, _CPR)])

    @pl.when(s == _NSUB - 1)
    def _zero_tail():
        pltpu.sync_copy(zeros_hbm.at[pl.ds(0, _TAIL)],
                        acc.at[pl.ds(_CPR * _NSUB, _TAIL)])

    plsc.subcore_barrier()

    nchunks = jnp.where(s == 0, _NCHUNKS // _NSUB + 1, _NCHUNKS // _NSUB)

    def body(t, carry):
        i = s + t * _NSUB
        pltpu.sync_copy(idx_hbm.at[c, i], idx_v)
        g0 = pltpu.async_copy(table.at[idx_v.at[2]], rows0, sem)
        g1 = pltpu.async_copy(table.at[idx_v.at[3]], rows1, sem)
        g0.wait()
        g1.wait()
        pltpu.sync_copy(rows0, acc.at[idx_v.at[0]], add=True)
        pltpu.sync_copy(rows1, acc.at[idx_v.at[1]], add=True)
        return carry

    lax.fori_loop(0, nchunks, body, 0)
    plsc.subcore_barrier()

    pltpu.sync_copy(acc.at[pl.ds(base, _CPR)],
                    out_hbm.at[c, pl.ds(base, _CPR)])

    @pl.when(s == _NSUB - 1)
    def _out_tail():
        pltpu.sync_copy(acc.at[pl.ds(_CPR * _NSUB, _TAIL)],
                        out_hbm.at[c, pl.ds(_CPR * _NSUB, _TAIL)])


def kernel(x, adj):
    n, d = x.shape
    row = adj[0]
    col = adj[1]

    xt2 = pl.pallas_call(
        _logmap_body,
        grid=(n // _BN,),
        in_specs=[pl.BlockSpec((_BN, d), lambda i: (i, 0))],
        out_specs=pl.BlockSpec((2, _BN, _HALF), lambda i: (0, i, 0)),
        out_shape=jax.ShapeDtypeStruct((2, n, _HALF), jnp.float32),
    )(x)
    table = xt2.reshape(2 * n, _HALF)

    rowc = row.reshape(_NCHUNKS, 2, _LANES)
    colc = col.reshape(_NCHUNKS, 2, _LANES)
    idx = jnp.stack([
        jnp.concatenate([rowc, colc], axis=1),
        jnp.concatenate([rowc, colc + n], axis=1),
    ])
    zeros = jnp.zeros((_CPR, _HALF), jnp.float32)

    mesh = plsc.VectorSubcoreMesh(core_axis_name="c", subcore_axis_name="s")
    support2 = pl.kernel(
        _sc_body,
        out_type=jax.ShapeDtypeStruct((2, n, _HALF), jnp.float32),
        mesh=mesh,
        scratch_types=[
            pltpu.VMEM((4, _LANES), jnp.int32),
            pltpu.VMEM((_LANES, _HALF), jnp.float32),
            pltpu.VMEM((_LANES, _HALF), jnp.float32),
            pltpu.VMEM_SHARED((n, _HALF), jnp.float32),
            pltpu.SemaphoreType.DMA,
        ],
    )(table, idx, zeros)

    out = pl.pallas_call(
        _expmap_body,
        grid=(n // _BN,),
        in_specs=[pl.BlockSpec((2, _BN, _HALF), lambda i: (0, i, 0))],
        out_specs=pl.BlockSpec((_BN, d), lambda i: (i, 0)),
        out_shape=jax.ShapeDtypeStruct((n, d), jnp.float32),
    )(support2)
    return out


# speedup vs baseline: 8.0707x; 1.3271x over previous
import jax
import jax.numpy as jnp
from jax import lax
from jax.experimental import pallas as pl
from jax.experimental.pallas import tpu as pltpu
from jax.experimental.pallas import tpu_sc as plsc

_MIN_NORM = 1e-15
_MAXNORM = 1.0 - 4e-3

_N, _E, _D = 10000, 160000, 256
_HALF = _D // 2
_LANES = 128
_CHUNK = _LANES
_NSUB = 16
_NHALF = 2
_CPH = 40
_CPS = _NHALF * _CPH
_NCHUNKS = _CPS * _NSUB
_EPAD = _NCHUNKS * _CHUNK
_PADROWS = 16
_CPR = (_N // _NSUB) // 8 * 8
_TAIL = _N - _CPR * _NSUB
_BN = 1000


def _logmap_body(x_ref, o_ref):
    x = x_ref[...]
    nrm = jnp.sqrt(jnp.sum(x * x, axis=1, keepdims=True))
    nrm = jnp.maximum(nrm, _MIN_NORM)
    t = jnp.clip(nrm, -1.0 + 1e-7, 1.0 - 1e-7)
    art = 0.5 * (jnp.log1p(t) - jnp.log1p(-t))
    xt = x * (art / nrm)
    o_ref[0] = xt[:, :_HALF]
    o_ref[1] = xt[:, _HALF:]


def _expmap_body(s_ref, o_ref):
    lo = s_ref[0]
    hi = s_ref[1]
    nrm = jnp.sqrt(jnp.sum(lo * lo, axis=1, keepdims=True)
                   + jnp.sum(hi * hi, axis=1, keepdims=True))
    nrm = jnp.maximum(nrm, _MIN_NORM)
    g = jnp.tanh(nrm) / nrm
    ylo = lo * g
    yhi = hi * g
    ynrm = jnp.sqrt(jnp.sum(ylo * ylo, axis=1, keepdims=True)
                    + jnp.sum(yhi * yhi, axis=1, keepdims=True))
    ynrm = jnp.maximum(ynrm, _MIN_NORM)
    scale = jnp.where(ynrm > _MAXNORM, _MAXNORM / ynrm, 1.0)
    o_ref[:, :_HALF] = ylo * scale
    o_ref[:, _HALF:] = yhi * scale


def _sc_body(table, idx_hbm, zeros_hbm, out_hbm,
             idx_all, ra, rb, acc,
             gsem_a, gsem_b, ssem_a, ssem_b):
    c = lax.axis_index("c")
    s = lax.axis_index("s")

    base = s * _CPR
    pltpu.sync_copy(zeros_hbm, acc.at[pl.ds(base, _CPR)])

    @pl.when(s == _NSUB - 1)
    def _zero_tail():
        pltpu.sync_copy(zeros_hbm.at[pl.ds(0, _TAIL)],
                        acc.at[pl.ds(_CPR * _NSUB, _TAIL)])

    plsc.subcore_barrier()

    def gather(t, r, sem):
        pltpu.async_copy(table.at[idx_all.at[t, 1]], r, sem)

    def wait_gather(t, r, sem):
        pltpu.make_async_copy(table.at[idx_all.at[t, 1]], r, sem).wait()

    def scatter(t, r, sem):
        pltpu.async_copy(r, acc.at[idx_all.at[t, 0]], sem, add=True)

    def wait_scatter(t, r, sem):
        pltpu.make_async_copy(r, acc.at[idx_all.at[t, 0]], sem).wait()

    def half(h, carry):
        pltpu.sync_copy(idx_hbm.at[c, s, h], idx_all)
        gather(0, ra, gsem_a)

        def pair(p, cc):
            ta = 2 * p
            tb = 2 * p + 1

            @pl.when(p > 0)
            def _():
                wait_scatter(tb, rb, ssem_b)
            gather(tb, rb, gsem_b)
            wait_gather(ta, ra, gsem_a)
            scatter(ta, ra, ssem_a)

            wait_scatter(ta, ra, ssem_a)

            @pl.when(p < _CPH // 2 - 1)
            def _():
                gather(ta + 2, ra, gsem_a)
            wait_gather(tb, rb, gsem_b)
            scatter(tb, rb, ssem_b)
            return cc

        lax.fori_loop(0, _CPH // 2, pair, 0)
        wait_scatter(_CPH - 1, rb, ssem_b)
        return carry

    lax.fori_loop(0, _NHALF, half, 0)
    plsc.subcore_barrier()

    pltpu.sync_copy(acc.at[pl.ds(base, _CPR)],
                    out_hbm.at[c, pl.ds(base, _CPR)])

    @pl.when(s == _NSUB - 1)
    def _out_tail():
        pltpu.sync_copy(acc.at[pl.ds(_CPR * _NSUB, _TAIL)],
                        out_hbm.at[c, pl.ds(_CPR * _NSUB, _TAIL)])


def kernel(x, adj):
    n, d = x.shape
    row = adj[0]
    col = adj[1]

    xt2 = pl.pallas_call(
        _logmap_body,
        grid=(n // _BN,),
        in_specs=[pl.BlockSpec((_BN, d), lambda i: (i, 0))],
        out_specs=pl.BlockSpec((2, _BN, _HALF), lambda i: (0, i, 0)),
        out_shape=jax.ShapeDtypeStruct((2, n, _HALF), jnp.float32),
    )(x)
    table = xt2.reshape(2 * n, _HALF)

    npad = _EPAD - _E
    k = jnp.arange(npad, dtype=jnp.int32)
    row_p = jnp.concatenate([row, n + (k % _PADROWS)])
    col_p = jnp.concatenate([col, (k * 97) % n])

    rowc = row_p.reshape(_NCHUNKS, 1, _LANES)
    colc = col_p.reshape(_NCHUNKS, 1, _LANES)
    idx = jnp.stack([
        jnp.concatenate([rowc, colc], axis=1),
        jnp.concatenate([rowc, colc + n], axis=1),
    ]).reshape(2, _NSUB, _NHALF, _CPH, 2, _LANES)
    zeros = jnp.zeros((_CPR, _HALF), jnp.float32)

    mesh = plsc.VectorSubcoreMesh(core_axis_name="c", subcore_axis_name="s")
    support2 = pl.kernel(
        _sc_body,
        out_type=jax.ShapeDtypeStruct((2, n, _HALF), jnp.float32),
        mesh=mesh,
        scratch_types=[
            pltpu.VMEM((_CPH, 2, _LANES), jnp.int32),
            pltpu.VMEM((_LANES, _HALF), jnp.float32),
            pltpu.VMEM((_LANES, _HALF), jnp.float32),
            pltpu.VMEM_SHARED((n + _PADROWS, _HALF), jnp.float32),
            pltpu.SemaphoreType.DMA,
            pltpu.SemaphoreType.DMA,
            pltpu.SemaphoreType.DMA,
            pltpu.SemaphoreType.DMA,
        ],
    )(table, idx, zeros)

    out = pl.pallas_call(
        _expmap_body,
        grid=(n // _BN,),
        in_specs=[pl.BlockSpec((2, _BN, _HALF), lambda i: (0, i, 0))],
        out_specs=pl.BlockSpec((_BN, d), lambda i: (i, 0)),
        out_shape=jax.ShapeDtypeStruct((n, d), jnp.float32),
    )(support2)
    return out


# speedup vs baseline: 8.4840x; 1.0512x over previous
import jax
import jax.numpy as jnp
from jax import lax
from jax.experimental import pallas as pl
from jax.experimental.pallas import tpu as pltpu
from jax.experimental.pallas import tpu_sc as plsc

_MIN_NORM = 1e-15
_MAXNORM = 1.0 - 4e-3

_N, _E, _D = 10000, 160000, 256
_HALF = _D // 2
_LANES = 128
_CHUNK = _LANES
_NSUB = 16
_NHALF = 2
_CPH = 40
_CPS = _NHALF * _CPH
_NCHUNKS = _CPS * _NSUB
_EPAD = _NCHUNKS * _CHUNK
_PADROWS = 16
_CPR = (_N // _NSUB) // 8 * 8
_TAIL = _N - _CPR * _NSUB
_BN = 2000
_CPB = _CPS


def _logmap_body(x_ref, o_ref):
    x = x_ref[...]
    nrm = jnp.sqrt(jnp.sum(x * x, axis=1, keepdims=True))
    nrm = jnp.maximum(nrm, _MIN_NORM)
    t = jnp.clip(nrm, -1.0 + 1e-7, 1.0 - 1e-7)
    art = 0.5 * (jnp.log1p(t) - jnp.log1p(-t))
    xt = x * (art / nrm)
    o_ref[0] = xt[:, :_HALF]
    o_ref[1] = xt[:, _HALF:]


def _expmap_body(s_ref, o_ref):
    lo = s_ref[0]
    hi = s_ref[1]
    nrm = jnp.sqrt(jnp.sum(lo * lo, axis=1, keepdims=True)
                   + jnp.sum(hi * hi, axis=1, keepdims=True))
    nrm = jnp.maximum(nrm, _MIN_NORM)
    g = jnp.tanh(nrm) / nrm
    ylo = lo * g
    yhi = hi * g
    ynrm = jnp.sqrt(jnp.sum(ylo * ylo, axis=1, keepdims=True)
                    + jnp.sum(yhi * yhi, axis=1, keepdims=True))
    ynrm = jnp.maximum(ynrm, _MIN_NORM)
    scale = jnp.where(ynrm > _MAXNORM, _MAXNORM / ynrm, 1.0)
    o_ref[:, :_HALF] = ylo * scale
    o_ref[:, _HALF:] = yhi * scale


def _idx_body(adj_ref, o_ref):
    s = pl.program_id(0)
    a = adj_ref[...]
    ht = lax.broadcasted_iota(jnp.int32, (_CPB, _LANES), 0)
    l = lax.broadcasted_iota(jnp.int32, (_CPB, _LANES), 1)
    eid = (s * _CPB + ht) * _LANES + l
    valid = eid < _E
    k = eid - _E
    dst = jnp.where(valid, a[0], _N + lax.rem(k, _PADROWS))
    src = jnp.where(valid, a[1], lax.rem(k * 97, _N))
    dst = dst.reshape(_NHALF, _CPH, _LANES)
    src = src.reshape(_NHALF, _CPH, _LANES)
    for h in range(_NHALF):
        o_ref[0, 0, h, 0] = dst[h]
        o_ref[0, 0, h, 1] = src[h]
        o_ref[1, 0, h, 0] = dst[h]
        o_ref[1, 0, h, 1] = src[h] + _N


def _sc_body(table, idx_hbm, out_hbm,
             idx_all, ra, rb, acc,
             gsem_a, gsem_b, ssem_a, ssem_b):
    c = lax.axis_index("c")
    s = lax.axis_index("s")

    def zfill(j, cc):
        for k in range(_HALF // 16):
            ra[j, pl.ds(k * 16, 16)] = jnp.zeros((16,), jnp.float32)
        return cc

    lax.fori_loop(0, _LANES, zfill, 0)
    base = s * _CPR
    for k in range(_CPR // _LANES):
        pltpu.sync_copy(ra, acc.at[pl.ds(base + k * _LANES, _LANES)])
    rem = _CPR % _LANES
    pltpu.sync_copy(ra.at[pl.ds(0, rem)],
                    acc.at[pl.ds(base + _CPR - rem, rem)])

    @pl.when(s == _NSUB - 1)
    def _zero_tail():
        pltpu.sync_copy(ra.at[pl.ds(0, _TAIL + _PADROWS)],
                        acc.at[pl.ds(_CPR * _NSUB, _TAIL + _PADROWS)])

    plsc.subcore_barrier()

    def gather(t, r, sem):
        pltpu.async_copy(table.at[idx_all.at[1, t]], r, sem)

    def wait_gather(t, r, sem):
        pltpu.make_async_copy(table.at[idx_all.at[1, t]], r, sem).wait()

    def scatter(t, r, sem):
        pltpu.async_copy(r, acc.at[idx_all.at[0, t]], sem, add=True)

    def wait_scatter(t, r, sem):
        pltpu.make_async_copy(r, acc.at[idx_all.at[0, t]], sem).wait()

    def half(h, carry):
        pltpu.sync_copy(idx_hbm.at[c, s, h], idx_all)
        gather(0, ra, gsem_a)

        def pair(p, cc):
            ta = 2 * p
            tb = 2 * p + 1

            @pl.when(p > 0)
            def _():
                wait_scatter(tb, rb, ssem_b)
            gather(tb, rb, gsem_b)
            wait_gather(ta, ra, gsem_a)
            scatter(ta, ra, ssem_a)

            wait_scatter(ta, ra, ssem_a)

            @pl.when(p < _CPH // 2 - 1)
            def _():
                gather(ta + 2, ra, gsem_a)
            wait_gather(tb, rb, gsem_b)
            scatter(tb, rb, ssem_b)
            return cc

        lax.fori_loop(0, _CPH // 2, pair, 0)
        wait_scatter(_CPH - 1, rb, ssem_b)
        return carry

    lax.fori_loop(0, _NHALF, half, 0)
    plsc.subcore_barrier()

    pltpu.sync_copy(acc.at[pl.ds(base, _CPR)],
                    out_hbm.at[c, pl.ds(base, _CPR)])

    @pl.when(s == _NSUB - 1)
    def _out_tail():
        pltpu.sync_copy(acc.at[pl.ds(_CPR * _NSUB, _TAIL)],
                        out_hbm.at[c, pl.ds(_CPR * _NSUB, _TAIL)])


def kernel(x, adj):
    n, d = x.shape
    xt2 = pl.pallas_call(
        _logmap_body,
        grid=(n // _BN,),
        in_specs=[pl.BlockSpec((_BN, d), lambda i: (i, 0))],
        out_specs=pl.BlockSpec((2, _BN, _HALF), lambda i: (0, i, 0)),
        out_shape=jax.ShapeDtypeStruct((2, n, _HALF), jnp.float32),
    )(x)
    table = xt2.reshape(2 * n, _HALF)

    adj_p = jnp.concatenate(
        [adj, jnp.zeros((2, _EPAD - _E), jnp.int32)], axis=1
    ).reshape(2, _NCHUNKS, _LANES)
    idx = pl.pallas_call(
        _idx_body,
        grid=(_NSUB,),
        in_specs=[pl.BlockSpec((2, _CPB, _LANES), lambda s: (0, s, 0))],
        out_specs=pl.BlockSpec((2, 1, _NHALF, 2, _CPH, _LANES),
                               lambda s: (0, s, 0, 0, 0, 0)),
        out_shape=jax.ShapeDtypeStruct(
            (2, _NSUB, _NHALF, 2, _CPH, _LANES), jnp.int32),
    )(adj_p)

    mesh = plsc.VectorSubcoreMesh(core_axis_name="c", subcore_axis_name="s")
    support2 = pl.kernel(
        _sc_body,
        out_type=jax.ShapeDtypeStruct((2, n, _HALF), jnp.float32),
        mesh=mesh,
        scratch_types=[
            pltpu.VMEM((2, _CPH, _LANES), jnp.int32),
            pltpu.VMEM((_LANES, _HALF), jnp.float32),
            pltpu.VMEM((_LANES, _HALF), jnp.float32),
            pltpu.VMEM_SHARED((n + _PADROWS, _HALF), jnp.float32),
            pltpu.SemaphoreType.DMA,
            pltpu.SemaphoreType.DMA,
            pltpu.SemaphoreType.DMA,
            pltpu.SemaphoreType.DMA,
        ],
    )(table, idx)

    out = pl.pallas_call(
        _expmap_body,
        grid=(n // _BN,),
        in_specs=[pl.BlockSpec((2, _BN, _HALF), lambda i: (0, i, 0))],
        out_specs=pl.BlockSpec((_BN, d), lambda i: (i, 0)),
        out_shape=jax.ShapeDtypeStruct((n, d), jnp.float32),
    )(support2)
    return out


# speedup vs baseline: 8.8452x; 1.0426x over previous
import jax
import jax.numpy as jnp
from jax import lax
from jax.experimental import pallas as pl
from jax.experimental.pallas import tpu as pltpu
from jax.experimental.pallas import tpu_sc as plsc

_MIN_NORM = 1e-15
_MAXNORM = 1.0 - 4e-3

_N, _E, _D = 10000, 160000, 256
_HALF = _D // 2
_LANES = 128
_CHUNK = _LANES
_NSUB = 16
_NHALF = 2
_CPH = 40
_CPS = _NHALF * _CPH
_NCHUNKS = _CPS * _NSUB
_EPAD = _NCHUNKS * _CHUNK
_PADROWS = 16
_CPR = (_N // _NSUB) // 8 * 8
_TAIL = _N - _CPR * _NSUB
_BN = 2000
_CPB = _CPS


def _logmap_body(x_ref, o_ref):
    x = x_ref[...]
    nrm = jnp.sqrt(jnp.sum(x * x, axis=1, keepdims=True))
    nrm = jnp.maximum(nrm, _MIN_NORM)
    t = jnp.clip(nrm, -1.0 + 1e-7, 1.0 - 1e-7)
    art = 0.5 * (jnp.log1p(t) - jnp.log1p(-t))
    xt = x * (art / nrm)
    o_ref[0] = xt[:, :_HALF]
    o_ref[1] = xt[:, _HALF:]


def _expmap_body(s_ref, o_ref):
    lo = s_ref[0]
    hi = s_ref[1]
    nrm = jnp.sqrt(jnp.sum(lo * lo, axis=1, keepdims=True)
                   + jnp.sum(hi * hi, axis=1, keepdims=True))
    nrm = jnp.maximum(nrm, _MIN_NORM)
    g = jnp.tanh(nrm) / nrm
    ylo = lo * g
    yhi = hi * g
    ynrm = jnp.sqrt(jnp.sum(ylo * ylo, axis=1, keepdims=True)
                    + jnp.sum(yhi * yhi, axis=1, keepdims=True))
    ynrm = jnp.maximum(ynrm, _MIN_NORM)
    scale = jnp.where(ynrm > _MAXNORM, _MAXNORM / ynrm, 1.0)
    o_ref[:, :_HALF] = ylo * scale
    o_ref[:, _HALF:] = yhi * scale


_IDXG = 2
_SPB = _NSUB // _IDXG


def _idx_body(adj_ref, o_ref):
    g = pl.program_id(0)
    a = adj_ref[...]
    nc = _SPB * _CPS
    ht = lax.broadcasted_iota(jnp.int32, (nc, _LANES), 0)
    l = lax.broadcasted_iota(jnp.int32, (nc, _LANES), 1)
    eid = (g * nc + ht) * _LANES + l
    valid = eid < _E
    k = eid - _E
    dst = jnp.where(valid, a[0], _N + lax.rem(k, _PADROWS))
    src = jnp.where(valid, a[1], lax.rem(k * 97, _N))
    dst = dst.reshape(_SPB, _NHALF, _CPH, _LANES)
    src = src.reshape(_SPB, _NHALF, _CPH, _LANES)
    for sub in range(_SPB):
        for h in range(_NHALF):
            o_ref[0, sub, h, 0] = dst[sub, h]
            o_ref[0, sub, h, 1] = src[sub, h]
            o_ref[1, sub, h, 0] = dst[sub, h]
            o_ref[1, sub, h, 1] = src[sub, h] + _N


def _sc_body(table, idx_hbm, out_hbm,
             idx_all, ra, rb, acc,
             gsem_a, gsem_b, ssem_a, ssem_b):
    c = lax.axis_index("c")
    s = lax.axis_index("s")

    def zfill(j, cc):
        for k in range(_HALF // 16):
            ra[j, pl.ds(k * 16, 16)] = jnp.zeros((16,), jnp.float32)
        return cc

    lax.fori_loop(0, _LANES, zfill, 0)
    base = s * _CPR
    for k in range(_CPR // _LANES):
        pltpu.sync_copy(ra, acc.at[pl.ds(base + k * _LANES, _LANES)])
    rem = _CPR % _LANES
    pltpu.sync_copy(ra.at[pl.ds(0, rem)],
                    acc.at[pl.ds(base + _CPR - rem, rem)])

    @pl.when(s == _NSUB - 1)
    def _zero_tail():
        pltpu.sync_copy(ra.at[pl.ds(0, _TAIL + _PADROWS)],
                        acc.at[pl.ds(_CPR * _NSUB, _TAIL + _PADROWS)])

    plsc.subcore_barrier()

    def gather(t, r, sem):
        pltpu.async_copy(table.at[idx_all.at[1, t]], r, sem)

    def wait_gather(t, r, sem):
        pltpu.make_async_copy(table.at[idx_all.at[1, t]], r, sem).wait()

    def scatter(t, r, sem):
        pltpu.async_copy(r, acc.at[idx_all.at[0, t]], sem, add=True)

    def wait_scatter(t, r, sem):
        pltpu.make_async_copy(r, acc.at[idx_all.at[0, t]], sem).wait()

    def half(h, carry):
        pltpu.sync_copy(idx_hbm.at[c, s, h], idx_all)
        gather(0, ra, gsem_a)

        def pair(p, cc):
            ta = 2 * p
            tb = 2 * p + 1

            @pl.when(p > 0)
            def _():
                wait_scatter(tb, rb, ssem_b)
            gather(tb, rb, gsem_b)
            wait_gather(ta, ra, gsem_a)
            scatter(ta, ra, ssem_a)

            wait_scatter(ta, ra, ssem_a)

            @pl.when(p < _CPH // 2 - 1)
            def _():
                gather(ta + 2, ra, gsem_a)
            wait_gather(tb, rb, gsem_b)
            scatter(tb, rb, ssem_b)
            return cc

        lax.fori_loop(0, _CPH // 2, pair, 0)
        wait_scatter(_CPH - 1, rb, ssem_b)
        return carry

    lax.fori_loop(0, _NHALF, half, 0)
    plsc.subcore_barrier()

    pltpu.sync_copy(acc.at[pl.ds(base, _CPR)],
                    out_hbm.at[c, pl.ds(base, _CPR)])

    @pl.when(s == _NSUB - 1)
    def _out_tail():
        pltpu.sync_copy(acc.at[pl.ds(_CPR * _NSUB, _TAIL)],
                        out_hbm.at[c, pl.ds(_CPR * _NSUB, _TAIL)])


def kernel(x, adj):
    n, d = x.shape
    xt2 = pl.pallas_call(
        _logmap_body,
        grid=(n // _BN,),
        in_specs=[pl.BlockSpec((_BN, d), lambda i: (i, 0))],
        out_specs=pl.BlockSpec((2, _BN, _HALF), lambda i: (0, i, 0)),
        out_shape=jax.ShapeDtypeStruct((2, n, _HALF), jnp.float32),
    )(x)
    table = xt2.reshape(2 * n, _HALF)

    idx = pl.pallas_call(
        _idx_body,
        grid=(_IDXG,),
        in_specs=[pl.BlockSpec((2, _SPB * _CPS, _LANES),
                               lambda g: (0, g, 0))],
        out_specs=pl.BlockSpec((2, _SPB, _NHALF, 2, _CPH, _LANES),
                               lambda g: (0, g, 0, 0, 0, 0)),
        out_shape=jax.ShapeDtypeStruct(
            (2, _NSUB, _NHALF, 2, _CPH, _LANES), jnp.int32),
    )(adj.reshape(2, _E // _LANES, _LANES))

    mesh = plsc.VectorSubcoreMesh(core_axis_name="c", subcore_axis_name="s")
    support2 = pl.kernel(
        _sc_body,
        out_type=jax.ShapeDtypeStruct((2, n, _HALF), jnp.float32),
        mesh=mesh,
        scratch_types=[
            pltpu.VMEM((2, _CPH, _LANES), jnp.int32),
            pltpu.VMEM((_LANES, _HALF), jnp.float32),
            pltpu.VMEM((_LANES, _HALF), jnp.float32),
            pltpu.VMEM_SHARED((n + _PADROWS, _HALF), jnp.float32),
            pltpu.SemaphoreType.DMA,
            pltpu.SemaphoreType.DMA,
            pltpu.SemaphoreType.DMA,
            pltpu.SemaphoreType.DMA,
        ],
    )(table, idx)

    out = pl.pallas_call(
        _expmap_body,
        grid=(n // _BN,),
        in_specs=[pl.BlockSpec((2, _BN, _HALF), lambda i: (0, i, 0))],
        out_specs=pl.BlockSpec((_BN, d), lambda i: (i, 0)),
        out_shape=jax.ShapeDtypeStruct((n, d), jnp.float32),
    )(support2)
    return out


# speedup vs baseline: 8.9184x; 1.0083x over previous
import jax
import jax.numpy as jnp
from jax import lax
from jax.experimental import pallas as pl
from jax.experimental.pallas import tpu as pltpu
from jax.experimental.pallas import tpu_sc as plsc

_MIN_NORM = 1e-15
_MAXNORM = 1.0 - 4e-3

_N, _E, _D = 10000, 160000, 256
_HALF = _D // 2
_LANES = 128
_CHUNK = _LANES
_NSUB = 16
_NHALF = 2
_CPH = 40
_CPS = _NHALF * _CPH
_NCHUNKS = _CPS * _NSUB
_EPAD = _NCHUNKS * _CHUNK
_PADROWS = 16
_CPR = (_N // _NSUB) // 8 * 8
_TAIL = _N - _CPR * _NSUB
_BN = 2000
_CPB = _CPS


def _logmap_body(x_ref, o_ref):
    x = x_ref[...]
    nrm = jnp.sqrt(jnp.sum(x * x, axis=1, keepdims=True))
    nrm = jnp.maximum(nrm, _MIN_NORM)
    t = jnp.clip(nrm, -1.0 + 1e-7, 1.0 - 1e-7)
    art = 0.5 * (jnp.log1p(t) - jnp.log1p(-t))
    xt = x * (art / nrm)
    o_ref[0] = xt[:, :_HALF]
    o_ref[1] = xt[:, _HALF:]


def _expmap_body(s_ref, o_ref):
    lo = s_ref[0]
    hi = s_ref[1]
    nrm = jnp.sqrt(jnp.sum(lo * lo, axis=1, keepdims=True)
                   + jnp.sum(hi * hi, axis=1, keepdims=True))
    nrm = jnp.maximum(nrm, _MIN_NORM)
    g = jnp.tanh(nrm) / nrm
    ylo = lo * g
    yhi = hi * g
    ynrm = jnp.sqrt(jnp.sum(ylo * ylo, axis=1, keepdims=True)
                    + jnp.sum(yhi * yhi, axis=1, keepdims=True))
    ynrm = jnp.maximum(ynrm, _MIN_NORM)
    scale = jnp.where(ynrm > _MAXNORM, _MAXNORM / ynrm, 1.0)
    o_ref[:, :_HALF] = ylo * scale
    o_ref[:, _HALF:] = yhi * scale


_IDXG = 2
_SPB = _NSUB // _IDXG


def _idx_body(adj_ref, o_ref):
    g = pl.program_id(0)
    a = adj_ref[...]
    nc = _SPB * _CPS
    ht = lax.broadcasted_iota(jnp.int32, (nc, _LANES), 0)
    l = lax.broadcasted_iota(jnp.int32, (nc, _LANES), 1)
    eid = (g * nc + ht) * _LANES + l
    valid = eid < _E
    k = eid - _E
    dst = jnp.where(valid, a[0], _N + lax.rem(k, _PADROWS))
    src = jnp.where(valid, a[1], lax.rem(k * 97, _N))
    dst = dst.reshape(_SPB, _NHALF, _CPH, _LANES)
    src = src.reshape(_SPB, _NHALF, _CPH, _LANES)
    for sub in range(_SPB):
        for h in range(_NHALF):
            o_ref[0, sub, h, 0] = dst[sub, h]
            o_ref[0, sub, h, 1] = src[sub, h]
            o_ref[1, sub, h, 0] = dst[sub, h]
            o_ref[1, sub, h, 1] = src[sub, h] + _N


def _sc_body(table, idx_hbm, out_hbm,
             idx_all, ra, rb, acc,
             gsem_a, gsem_b, ssem_a, ssem_b):
    c = lax.axis_index("c")
    s = lax.axis_index("s")

    def gather(t, r, sem):
        pltpu.async_copy(table.at[idx_all.at[1, t]], r, sem)

    def wait_gather(t, r, sem):
        pltpu.make_async_copy(table.at[idx_all.at[1, t]], r, sem).wait()

    def scatter(t, r, sem):
        pltpu.async_copy(r, acc.at[idx_all.at[0, t]], sem, add=True)

    def wait_scatter(t, r, sem):
        pltpu.make_async_copy(r, acc.at[idx_all.at[0, t]], sem).wait()

    pltpu.sync_copy(idx_hbm.at[c, s, 0], idx_all)
    gather(0, ra, gsem_a)

    def zfill(j, cc):
        for jj in range(16):
            for k in range(_HALF // 16):
                rb[j * 16 + jj, pl.ds(k * 16, 16)] = jnp.zeros(
                    (16,), jnp.float32)
        return cc

    lax.fori_loop(0, _LANES // 16, zfill, 0)
    base = s * _CPR
    for k in range(_CPR // _LANES):
        pltpu.sync_copy(rb, acc.at[pl.ds(base + k * _LANES, _LANES)])
    rem = _CPR % _LANES
    pltpu.sync_copy(rb.at[pl.ds(0, rem)],
                    acc.at[pl.ds(base + _CPR - rem, rem)])

    @pl.when(s == _NSUB - 1)
    def _zero_tail():
        pltpu.sync_copy(rb.at[pl.ds(0, _TAIL + _PADROWS)],
                        acc.at[pl.ds(_CPR * _NSUB, _TAIL + _PADROWS)])

    gather(1, rb, gsem_b)
    plsc.subcore_barrier()

    def half_pass(h, double_primed):
        if not double_primed:
            pltpu.sync_copy(idx_hbm.at[c, s, h], idx_all)
            gather(0, ra, gsem_a)

        def pair(p, cc):
            ta = 2 * p
            tb = 2 * p + 1

            @pl.when(p > 0)
            def _():
                wait_scatter(tb, rb, ssem_b)
            if double_primed:
                @pl.when(p > 0)
                def _():
                    gather(tb, rb, gsem_b)
            else:
                gather(tb, rb, gsem_b)
            wait_gather(ta, ra, gsem_a)
            scatter(ta, ra, ssem_a)

            wait_scatter(ta, ra, ssem_a)

            @pl.when(p < _CPH // 2 - 1)
            def _():
                gather(ta + 2, ra, gsem_a)
            wait_gather(tb, rb, gsem_b)
            scatter(tb, rb, ssem_b)
            return cc

        lax.fori_loop(0, _CPH // 2, pair, 0)
        wait_scatter(_CPH - 1, rb, ssem_b)

    half_pass(0, True)
    half_pass(1, False)
    plsc.subcore_barrier()

    pltpu.sync_copy(acc.at[pl.ds(base, _CPR)],
                    out_hbm.at[c, pl.ds(base, _CPR)])

    @pl.when(s == _NSUB - 1)
    def _out_tail():
        pltpu.sync_copy(acc.at[pl.ds(_CPR * _NSUB, _TAIL)],
                        out_hbm.at[c, pl.ds(_CPR * _NSUB, _TAIL)])


def kernel(x, adj):
    n, d = x.shape
    xt2 = pl.pallas_call(
        _logmap_body,
        grid=(n // _BN,),
        in_specs=[pl.BlockSpec((_BN, d), lambda i: (i, 0))],
        out_specs=pl.BlockSpec((2, _BN, _HALF), lambda i: (0, i, 0)),
        out_shape=jax.ShapeDtypeStruct((2, n, _HALF), jnp.float32),
    )(x)
    table = xt2.reshape(2 * n, _HALF)

    idx = pl.pallas_call(
        _idx_body,
        grid=(_IDXG,),
        in_specs=[pl.BlockSpec((2, _SPB * _CPS, _LANES),
                               lambda g: (0, g, 0))],
        out_specs=pl.BlockSpec((2, _SPB, _NHALF, 2, _CPH, _LANES),
                               lambda g: (0, g, 0, 0, 0, 0)),
        out_shape=jax.ShapeDtypeStruct(
            (2, _NSUB, _NHALF, 2, _CPH, _LANES), jnp.int32),
    )(adj.reshape(2, _E // _LANES, _LANES))

    mesh = plsc.VectorSubcoreMesh(core_axis_name="c", subcore_axis_name="s")
    support2 = pl.kernel(
        _sc_body,
        out_type=jax.ShapeDtypeStruct((2, n, _HALF), jnp.float32),
        mesh=mesh,
        scratch_types=[
            pltpu.VMEM((2, _CPH, _LANES), jnp.int32),
            pltpu.VMEM((_LANES, _HALF), jnp.float32),
            pltpu.VMEM((_LANES, _HALF), jnp.float32),
            pltpu.VMEM_SHARED((n + _PADROWS, _HALF), jnp.float32),
            pltpu.SemaphoreType.DMA,
            pltpu.SemaphoreType.DMA,
            pltpu.SemaphoreType.DMA,
            pltpu.SemaphoreType.DMA,
        ],
    )(table, idx)

    out = pl.pallas_call(
        _expmap_body,
        grid=(n // _BN,),
        in_specs=[pl.BlockSpec((2, _BN, _HALF), lambda i: (0, i, 0))],
        out_specs=pl.BlockSpec((_BN, d), lambda i: (i, 0)),
        out_shape=jax.ShapeDtypeStruct((n, d), jnp.float32),
    )(support2)
    return out


# speedup vs baseline: 8.9410x; 1.0025x over previous
import jax
import jax.numpy as jnp
from jax import lax
from jax.experimental import pallas as pl
from jax.experimental.pallas import tpu as pltpu
from jax.experimental.pallas import tpu_sc as plsc

_MIN_NORM = 1e-15
_MAXNORM = 1.0 - 4e-3

_N, _E, _D = 10000, 160000, 256
_HALF = _D // 2
_LANES = 128
_CHUNK = _LANES
_NSUB = 16
_NHALF = 2
_CPH = 40
_CPS = _NHALF * _CPH
_CPR = (_N // _NSUB) // 8 * 8
_TAIL = _N - _CPR * _NSUB
_BN = 2000


def _logmap_body(x_ref, o_ref):
    x = x_ref[...]
    nrm = jnp.sqrt(jnp.sum(x * x, axis=1, keepdims=True))
    nrm = jnp.maximum(nrm, _MIN_NORM)
    t = jnp.clip(nrm, -1.0 + 1e-7, 1.0 - 1e-7)
    art = 0.5 * (jnp.log1p(t) - jnp.log1p(-t))
    xt = x * (art / nrm)
    o_ref[0] = xt[:, :_HALF]
    o_ref[1] = xt[:, _HALF:]


def _expmap_body(s_ref, o_ref):
    lo = s_ref[0]
    hi = s_ref[1]
    nrm = jnp.sqrt(jnp.sum(lo * lo, axis=1, keepdims=True)
                   + jnp.sum(hi * hi, axis=1, keepdims=True))
    nrm = jnp.maximum(nrm, _MIN_NORM)
    g = jnp.tanh(nrm) / nrm
    ylo = lo * g
    yhi = hi * g
    ynrm = jnp.sqrt(jnp.sum(ylo * ylo, axis=1, keepdims=True)
                    + jnp.sum(yhi * yhi, axis=1, keepdims=True))
    ynrm = jnp.maximum(ynrm, _MIN_NORM)
    scale = jnp.where(ynrm > _MAXNORM, _MAXNORM / ynrm, 1.0)
    o_ref[:, :_HALF] = ylo * scale
    o_ref[:, _HALF:] = yhi * scale


_PADROWS = 8
_NCHUNKS = (_E // _LANES + 7) // 8 * 8
_EPAD = _NCHUNKS * _LANES - _E
_SHORT = _NCHUNKS - (_NSUB - 1) * _CPS - _CPH


def _sc_body(table, adj_hbm, out_hbm,
             idx_row, idx_col, ra, rb, acc,
             gsem_a, gsem_b, ssem_a, ssem_b):
    c = lax.axis_index("c")
    s = lax.axis_index("s")
    tbl = table.at[c]

    def gather(t, r, sem):
        pltpu.async_copy(tbl.at[idx_col.at[t]], r, sem)

    def wait_gather(t, r, sem):
        pltpu.make_async_copy(tbl.at[idx_col.at[t]], r, sem).wait()

    def scatter(t, r, sem):
        pltpu.async_copy(r, acc.at[idx_row.at[t]], sem, add=True)

    def wait_scatter(t, r, sem):
        pltpu.make_async_copy(r, acc.at[idx_row.at[t]], sem).wait()

    def fetch_idx(h, cnt):
        off = pl.multiple_of(s * _CPS + h * _CPH, 8)
        pltpu.sync_copy(adj_hbm.at[0, pl.ds(off, cnt)],
                        idx_row.at[pl.ds(0, cnt)])
        pltpu.sync_copy(adj_hbm.at[1, pl.ds(off, cnt)],
                        idx_col.at[pl.ds(0, cnt)])

    fetch_idx(0, _CPH)
    gather(0, ra, gsem_a)

    def zfill(j, cc):
        for jj in range(16):
            for k in range(_HALF // 16):
                rb[j * 16 + jj, pl.ds(k * 16, 16)] = jnp.zeros(
                    (16,), jnp.float32)
        return cc

    lax.fori_loop(0, _LANES // 16, zfill, 0)
    base = s * _CPR
    for k in range(_CPR // _LANES):
        pltpu.sync_copy(rb, acc.at[pl.ds(base + k * _LANES, _LANES)])
    rem = _CPR % _LANES
    pltpu.sync_copy(rb.at[pl.ds(0, rem)],
                    acc.at[pl.ds(base + _CPR - rem, rem)])

    @pl.when(s == _NSUB - 1)
    def _zero_tail():
        pltpu.sync_copy(rb.at[pl.ds(0, _TAIL + _PADROWS)],
                        acc.at[pl.ds(_CPR * _NSUB, _TAIL + _PADROWS)])

    gather(1, rb, gsem_b)
    plsc.subcore_barrier()

    def half_pass(h, double_primed, ncnk):
        if not double_primed:
            fetch_idx(h, ncnk)
            gather(0, ra, gsem_a)

        def pair(p, cc):
            ta = 2 * p
            tb = 2 * p + 1

            @pl.when(p > 0)
            def _():
                wait_scatter(tb, rb, ssem_b)
            if double_primed:
                @pl.when(p > 0)
                def _():
                    gather(tb, rb, gsem_b)
            else:
                gather(tb, rb, gsem_b)
            wait_gather(ta, ra, gsem_a)
            scatter(ta, ra, ssem_a)

            wait_scatter(ta, ra, ssem_a)

            @pl.when(p < ncnk // 2 - 1)
            def _():
                gather(ta + 2, ra, gsem_a)
            wait_gather(tb, rb, gsem_b)
            scatter(tb, rb, ssem_b)
            return cc

        lax.fori_loop(0, ncnk // 2, pair, 0)
        wait_scatter(ncnk - 1, rb, ssem_b)

    half_pass(0, True, _CPH)

    @pl.when(s < _NSUB - 1)
    def _full_second():
        half_pass(1, False, _CPH)

    @pl.when(s == _NSUB - 1)
    def _short_second():
        half_pass(1, False, _SHORT)

    plsc.subcore_barrier()

    pltpu.sync_copy(acc.at[pl.ds(base, _CPR)],
                    out_hbm.at[c, pl.ds(base, _CPR)])

    @pl.when(s == _NSUB - 1)
    def _out_tail():
        pltpu.sync_copy(acc.at[pl.ds(_CPR * _NSUB, _TAIL)],
                        out_hbm.at[c, pl.ds(_CPR * _NSUB, _TAIL)])


def kernel(x, adj):
    n, d = x.shape
    xt2 = pl.pallas_call(
        _logmap_body,
        grid=(n // _BN,),
        in_specs=[pl.BlockSpec((_BN, d), lambda i: (i, 0))],
        out_specs=pl.BlockSpec((2, _BN, _HALF), lambda i: (0, i, 0)),
        out_shape=jax.ShapeDtypeStruct((2, n, _HALF), jnp.float32),
    )(x)

    ar = jnp.arange(_EPAD, dtype=jnp.int32)
    adj_p = jnp.concatenate(
        [adj, jnp.stack([n + lax.rem(ar, _PADROWS), ar])], axis=1)

    mesh = plsc.VectorSubcoreMesh(core_axis_name="c", subcore_axis_name="s")
    support2 = pl.kernel(
        _sc_body,
        out_type=jax.ShapeDtypeStruct((2, n, _HALF), jnp.float32),
        mesh=mesh,
        scratch_types=[
            pltpu.VMEM((_CPH, _LANES), jnp.int32),
            pltpu.VMEM((_CPH, _LANES), jnp.int32),
            pltpu.VMEM((_LANES, _HALF), jnp.float32),
            pltpu.VMEM((_LANES, _HALF), jnp.float32),
            pltpu.VMEM_SHARED((n + _PADROWS, _HALF), jnp.float32),
            pltpu.SemaphoreType.DMA,
            pltpu.SemaphoreType.DMA,
            pltpu.SemaphoreType.DMA,
            pltpu.SemaphoreType.DMA,
        ],
    )(xt2, adj_p.reshape(2, _NCHUNKS, _LANES))

    out = pl.pallas_call(
        _expmap_body,
        grid=(n // _BN,),
        in_specs=[pl.BlockSpec((2, _BN, _HALF), lambda i: (0, i, 0))],
        out_specs=pl.BlockSpec((_BN, d), lambda i: (i, 0)),
        out_shape=jax.ShapeDtypeStruct((n, d), jnp.float32),
    )(support2)
    return out
